# baseline (speedup 1.0000x reference)
import jax
import jax.numpy as jnp
from jax.experimental import pallas as pl

N_ATOMS = 100000
EMB = 128
N_MOL = 256
BLOCK = 10000
N_STEPS = 5


def _seg_contrib(e_ref, ids_ref, col, w_bf, b_val):
    v = jnp.dot(e_ref[:].astype(jnp.bfloat16), w_bf,
                preferred_element_type=jnp.float32)
    v = v + b_val
    ids_col = ids_ref[0]
    oh = jnp.where(ids_col == col, jnp.bfloat16(1), jnp.bfloat16(0))
    return jax.lax.dot_general(
        v.reshape(1, BLOCK).astype(jnp.bfloat16), oh,
        (((1,), (0,)), ((), ())),
        preferred_element_type=jnp.float32,
    )


def _head_kernel(ea_ref, eb_ref, ia_ref, ib_ref, col_ref, w_ref, b_ref, out_ref):
    i = pl.program_id(0)
    w_bf = w_ref[:].astype(jnp.bfloat16)
    b_val = b_ref[0, 0]
    col = col_ref[:]
    contrib = (_seg_contrib(ea_ref, ia_ref, col, w_bf, b_val)
               + _seg_contrib(eb_ref, ib_ref, col, w_bf, b_val))

    @pl.when(i == 0)
    def _():
        out_ref[:] = jnp.zeros_like(out_ref)

    out_ref[:] += contrib


def kernel(energy, batch, W, b):
    ids_col = batch.astype(jnp.int32).astype(jnp.bfloat16).reshape(
        2 * N_STEPS, BLOCK, 1)
    col_row = jnp.arange(N_MOL, dtype=jnp.int32).astype(jnp.bfloat16).reshape(
        1, N_MOL)
    b2d = b.reshape(1, 1)
    out = pl.pallas_call(
        _head_kernel,
        grid=(N_STEPS,),
        in_specs=[
            pl.BlockSpec((BLOCK, EMB), lambda i: (i, 0)),
            pl.BlockSpec((BLOCK, EMB), lambda i: (i + N_STEPS, 0)),
            pl.BlockSpec((1, BLOCK, 1), lambda i: (i, 0, 0)),
            pl.BlockSpec((1, BLOCK, 1), lambda i: (i + N_STEPS, 0, 0)),
            pl.BlockSpec((1, N_MOL), lambda i: (0, 0)),
            pl.BlockSpec((EMB, 1), lambda i: (0, 0)),
            pl.BlockSpec((1, 1), lambda i: (0, 0)),
        ],
        out_specs=pl.BlockSpec((1, N_MOL), lambda i: (0, 0)),
        out_shape=jax.ShapeDtypeStruct((1, N_MOL), jnp.float32),
    )(energy, energy, ids_col, ids_col, col_row, W, b2d)
    return out[0]


# speedup vs baseline: 2.5221x; 2.5221x over previous
import jax
import jax.numpy as jnp
from jax.experimental import pallas as pl

N_ATOMS = 100000
EMB = 128
N_MOL = 256
BLOCK = 10000
N_STEPS = 5


def _seg_contrib(e_ref, ids_ref, colc, w_bf, b_val):
    v = jnp.dot(e_ref[:].astype(jnp.bfloat16), w_bf,
                preferred_element_type=jnp.float32)
    v = v + b_val
    ids_row = ids_ref[0]
    oht = jnp.where(colc == ids_row, jnp.bfloat16(1), jnp.bfloat16(0))
    return jax.lax.dot_general(
        oht, v.astype(jnp.bfloat16),
        (((1,), (0,)), ((), ())),
        preferred_element_type=jnp.float32,
    )


def _head_kernel(ea_ref, eb_ref, ia_ref, ib_ref, colc_ref, w_ref, b_ref,
                 out_ref):
    i = pl.program_id(0)
    w_bf = w_ref[:].astype(jnp.bfloat16)
    b_val = b_ref[0, 0]
    colc = colc_ref[:]
    contrib = (_seg_contrib(ea_ref, ia_ref, colc, w_bf, b_val)
               + _seg_contrib(eb_ref, ib_ref, colc, w_bf, b_val))

    @pl.when(i == 0)
    def _():
        out_ref[:] = jnp.zeros_like(out_ref)

    out_ref[:] += contrib


def kernel(energy, batch, W, b):
    ids_row = batch.astype(jnp.int32).astype(jnp.bfloat16).reshape(
        2 * N_STEPS, 1, BLOCK)
    col_col = jnp.arange(N_MOL, dtype=jnp.int32).astype(jnp.bfloat16).reshape(
        N_MOL, 1)
    b2d = b.reshape(1, 1)
    out = pl.pallas_call(
        _head_kernel,
        grid=(N_STEPS,),
        in_specs=[
            pl.BlockSpec((BLOCK, EMB), lambda i: (i, 0)),
            pl.BlockSpec((BLOCK, EMB), lambda i: (i + N_STEPS, 0)),
            pl.BlockSpec((1, 1, BLOCK), lambda i: (i, 0, 0)),
            pl.BlockSpec((1, 1, BLOCK), lambda i: (i + N_STEPS, 0, 0)),
            pl.BlockSpec((N_MOL, 1), lambda i: (0, 0)),
            pl.BlockSpec((EMB, 1), lambda i: (0, 0)),
            pl.BlockSpec((1, 1), lambda i: (0, 0)),
        ],
        out_specs=pl.BlockSpec((N_MOL, 1), lambda i: (0, 0)),
        out_shape=jax.ShapeDtypeStruct((N_MOL, 1), jnp.float32),
    )(energy, energy, ids_row, ids_row, col_col, W, b2d)
    return out[:, 0]


# speedup vs baseline: 2.7580x; 1.0935x over previous
import jax
import jax.numpy as jnp
from jax.experimental import pallas as pl

N_ATOMS = 100000
EMB = 128
N_MOL = 256
BLOCK = 5000
N_STEPS = 10


def _seg_contrib(e_ref, ids_ref, w_bf, b_val):
    v = jnp.dot(e_ref[:].astype(jnp.bfloat16), w_bf,
                preferred_element_type=jnp.float32)
    v = v + b_val
    ids = ids_ref[0, 0, :]
    col = jax.lax.broadcasted_iota(jnp.int32, (BLOCK, N_MOL), 1)
    oh = (ids[:, None] == col).astype(jnp.float32)
    return jax.lax.dot_general(
        v.reshape(1, BLOCK), oh,
        (((1,), (0,)), ((), ())),
        preferred_element_type=jnp.float32,
    )


def _head_kernel(ea_ref, eb_ref, ia_ref, ib_ref, w_ref, b_ref, out_ref):
    i = pl.program_id(0)
    w_bf = w_ref[:].astype(jnp.bfloat16)
    b_val = b_ref[0, 0]
    contrib = (_seg_contrib(ea_ref, ia_ref, w_bf, b_val)
               + _seg_contrib(eb_ref, ib_ref, w_bf, b_val))

    @pl.when(i == 0)
    def _():
        out_ref[:] = jnp.zeros_like(out_ref)

    out_ref[:] += contrib


def kernel(energy, batch, W, b):
    ids3d = batch.astype(jnp.int32).reshape(2 * N_STEPS, 1, BLOCK)
    b2d = b.reshape(1, 1)
    out = pl.pallas_call(
        _head_kernel,
        grid=(N_STEPS,),
        in_specs=[
            pl.BlockSpec((BLOCK, EMB), lambda i: (i, 0)),
            pl.BlockSpec((BLOCK, EMB), lambda i: (i + N_STEPS, 0)),
            pl.BlockSpec((1, 1, BLOCK), lambda i: (i, 0, 0)),
            pl.BlockSpec((1, 1, BLOCK), lambda i: (i + N_STEPS, 0, 0)),
            pl.BlockSpec((EMB, 1), lambda i: (0, 0)),
            pl.BlockSpec((1, 1), lambda i: (0, 0)),
        ],
        out_specs=pl.BlockSpec((1, N_MOL), lambda i: (0, 0)),
        out_shape=jax.ShapeDtypeStruct((1, N_MOL), jnp.float32),
    )(energy, energy, ids3d, ids3d, W, b2d)
    return out[0]
